# baseline (speedup 1.0000x reference)
import functools

import jax
import jax.numpy as jnp
from jax import lax
from jax.experimental import pallas as pl
from jax.experimental.pallas import tpu as pltpu
from jax.experimental.pallas import tpu_sc as plsc

_B = 16384
_C = 128
_F = 512
_NC = 2
_NS = 16
_NW = _NC * _NS
_RPW = _B // _NW
_TILE_R = 32
_NT = _RPW // _TILE_R
_NB = 4
_NQ = _NT // _NB
_LANES = 16
_GROUPS = _C // _LANES

_mesh = plsc.VectorSubcoreMesh(core_axis_name="c", subcore_axis_name="s")


@functools.partial(
    pl.kernel,
    mesh=_mesh,
    out_type=jax.ShapeDtypeStruct((_B * _F,), jnp.float32),
    compiler_params=pltpu.CompilerParams(needs_layout_passes=False),
    scratch_types=[
        pltpu.VMEM((_C,), jnp.int32),
        [pltpu.VMEM((_TILE_R * _C,), jnp.float32)] * _NB,
        [pltpu.VMEM((_TILE_R * _F,), jnp.float32)] * _NB,
        [pltpu.SemaphoreType.DMA] * _NB,
        [pltpu.SemaphoreType.DMA] * _NB,
    ],
)
def _scatter_kernel(x_hbm, sel_hbm, out_hbm, sel_v, x_vs, o_vs, sxs, sos):
    wid = lax.axis_index("s") * _NC + lax.axis_index("c")
    base = wid * _RPW

    pltpu.sync_copy(sel_hbm, sel_v)

    def _x_slice(t):
        return x_hbm.at[pl.ds((base + t * _TILE_R) * _C, _TILE_R * _C)]

    def _o_slice(t):
        return out_hbm.at[pl.ds((base + t * _TILE_R) * _F, _TILE_R * _F)]

    for b in range(_NB):
        pltpu.async_copy(_x_slice(b), x_vs[b], sxs[b])

    zeros = jnp.zeros((_LANES,), jnp.float32)

    @plsc.parallel_loop(0, _TILE_R * _F // _LANES, unroll=8)
    def _zero_chunk(i):
        for b in range(_NB):
            o_vs[b][pl.ds(i * _LANES, _LANES)] = zeros

    sel_groups = [sel_v[pl.ds(g * _LANES, _LANES)] for g in range(_GROUPS)]

    def _scatter_tile(x_ref, o_ref):
        @plsc.parallel_loop(0, _TILE_R, unroll=8)
        def _row(r):
            rbase = jnp.full((_LANES,), r * _F, jnp.int32)
            for g in range(_GROUPS):
                v = x_ref[pl.ds(r * _C + g * _LANES, _LANES)]
                plsc.store_scatter(o_ref, [sel_groups[g] + rbase], v)

    def _round(i, carry):
        for b in range(_NB):
            t = _NB * i + b
            pltpu.make_async_copy(_x_slice(t), x_vs[b], sxs[b]).wait()

            @pl.when(i > 0)
            def _wait_o():
                pltpu.make_async_copy(o_vs[b], _o_slice(t), sos[b]).wait()

            _scatter_tile(x_vs[b], o_vs[b])
            pltpu.async_copy(o_vs[b], _o_slice(t), sos[b])

            @pl.when(i < _NQ - 1)
            def _start_next_x():
                pltpu.async_copy(_x_slice(t + _NB), x_vs[b], sxs[b])
        return carry

    lax.fori_loop(0, _NQ, _round, 0)

    for b in range(_NB):
        pltpu.make_async_copy(o_vs[b], _o_slice(b), sos[b]).wait()


def kernel(x, sel):
    out_flat = _scatter_kernel(x.reshape(_B * _C), sel)
    return out_flat.reshape(_B, _F)


# speedup vs baseline: 1.0229x; 1.0229x over previous
import functools

import jax
import jax.numpy as jnp
from jax import lax
from jax.experimental import pallas as pl
from jax.experimental.pallas import tpu as pltpu
from jax.experimental.pallas import tpu_sc as plsc

_B = 16384
_C = 128
_F = 512
_NC = 2
_NS = 16
_NW = _NC * _NS
_RPW = _B // _NW
_TILE_R = 64
_NT = _RPW // _TILE_R
_NPAIR = _NT // 2
_LANES = 16
_GROUPS = _C // _LANES

_mesh = plsc.VectorSubcoreMesh(core_axis_name="c", subcore_axis_name="s")


@functools.partial(
    pl.kernel,
    mesh=_mesh,
    out_type=jax.ShapeDtypeStruct((_B * _F,), jnp.float32),
    compiler_params=pltpu.CompilerParams(needs_layout_passes=False),
    scratch_types=[
        pltpu.VMEM((_C,), jnp.int32),
        pltpu.VMEM((_TILE_R * _C,), jnp.float32),
        pltpu.VMEM((_TILE_R * _C,), jnp.float32),
        pltpu.VMEM((_TILE_R * _F,), jnp.float32),
        pltpu.VMEM((_TILE_R * _F,), jnp.float32),
        pltpu.SemaphoreType.DMA,
        pltpu.SemaphoreType.DMA,
        pltpu.SemaphoreType.DMA,
        pltpu.SemaphoreType.DMA,
    ],
)
def _scatter_kernel(x_hbm, sel_hbm, out_hbm, sel_v, x_v0, x_v1, o_v0, o_v1,
                    sx0, sx1, so0, so1):
    wid = lax.axis_index("s") * _NC + lax.axis_index("c")
    base = wid * _RPW

    def _x_slice(t):
        return x_hbm.at[pl.ds((base + t * _TILE_R) * _C, _TILE_R * _C)]

    def _o_slice(t):
        return out_hbm.at[pl.ds((base + t * _TILE_R) * _F, _TILE_R * _F)]

    pltpu.async_copy(_x_slice(0), x_v0, sx0)
    pltpu.async_copy(_x_slice(1), x_v1, sx1)

    pltpu.sync_copy(sel_hbm, sel_v)

    zeros = jnp.zeros((_LANES,), jnp.float32)

    def _zero_buf(o_ref):
        @plsc.parallel_loop(0, _TILE_R * _F // _LANES, unroll=8)
        def _zero_chunk(i):
            o_ref[pl.ds(i * _LANES, _LANES)] = zeros

    sel_groups = [sel_v[pl.ds(g * _LANES, _LANES)] for g in range(_GROUPS)]

    def _scatter_tile(x_ref, o_ref):
        @plsc.parallel_loop(0, _TILE_R, unroll=8)
        def _row(r):
            rbase = jnp.full((_LANES,), r * _F, jnp.int32)
            for g in range(_GROUPS):
                v = x_ref[pl.ds(r * _C + g * _LANES, _LANES)]
                plsc.store_scatter(o_ref, [sel_groups[g] + rbase], v)

    _zero_buf(o_v0)
    pltpu.make_async_copy(_x_slice(0), x_v0, sx0).wait()
    _scatter_tile(x_v0, o_v0)
    pltpu.async_copy(o_v0, _o_slice(0), so0)
    pltpu.async_copy(_x_slice(2), x_v0, sx0)

    _zero_buf(o_v1)
    pltpu.make_async_copy(_x_slice(1), x_v1, sx1).wait()
    _scatter_tile(x_v1, o_v1)
    pltpu.async_copy(o_v1, _o_slice(1), so1)
    pltpu.async_copy(_x_slice(3), x_v1, sx1)

    def _pair(i, carry):
        t0 = 2 * i
        pltpu.make_async_copy(_x_slice(t0), x_v0, sx0).wait()
        pltpu.make_async_copy(o_v0, _o_slice(t0), so0).wait()
        _scatter_tile(x_v0, o_v0)
        pltpu.async_copy(o_v0, _o_slice(t0), so0)

        @pl.when(i < _NPAIR - 1)
        def _refill_x0():
            pltpu.async_copy(_x_slice(t0 + 2), x_v0, sx0)

        pltpu.make_async_copy(_x_slice(t0 + 1), x_v1, sx1).wait()
        pltpu.make_async_copy(o_v1, _o_slice(t0 + 1), so1).wait()
        _scatter_tile(x_v1, o_v1)
        pltpu.async_copy(o_v1, _o_slice(t0 + 1), so1)

        @pl.when(i < _NPAIR - 1)
        def _refill_x1():
            pltpu.async_copy(_x_slice(t0 + 3), x_v1, sx1)

        return carry

    lax.fori_loop(1, _NPAIR, _pair, 0)

    pltpu.make_async_copy(o_v0, _o_slice(0), so0).wait()
    pltpu.make_async_copy(o_v1, _o_slice(1), so1).wait()


def kernel(x, sel):
    out_flat = _scatter_kernel(x.reshape(_B * _C), sel)
    return out_flat.reshape(_B, _F)
